# Initial kernel scaffold; baseline (speedup 1.0000x reference)
import jax
import jax.numpy as jnp
from jax.experimental import pallas as pl


def kernel(prediction):
    raise NotImplementedError("write your pallas kernel here")


# speedup vs baseline: 140.6341x; 140.6341x over previous
import functools

import numpy as np
import jax
import jax.numpy as jnp
from jax import lax
from jax.experimental import pallas as pl
from jax.experimental.pallas import tpu as pltpu
from jax.experimental.pallas import tpu_sc as plsc

H = W = 512
B, C = 4, 2
HW = H * W
DENSITY = 0.05
KAPPA = 16
REG_WEIGHT = 0.001

NW = 32
CH = 2048


def _static_coords():
    rng = np.random.RandomState(0)
    num_anchors = int(DENSITY * H * W)
    ay = rng.randint(KAPPA, H - KAPPA, num_anchors)
    ax = rng.randint(KAPPA, W - KAPPA, num_anchors)
    anchors = np.stack((ax, ay), axis=1)
    num_refs = int(DENSITY * np.pi * KAPPA ** 2)
    anchors = np.repeat(anchors, num_refs, axis=0)
    n = len(anchors)
    theta = 2.0 * np.pi * rng.random_sample(n)
    r = KAPPA * rng.random_sample(n)
    offsets = np.stack((r * np.cos(theta), r * np.sin(theta)), axis=1)
    refs = (anchors + offsets).astype(np.int64)
    return anchors.astype(np.int64), refs


_anchors, _refs = _static_coords()
N = len(_anchors)
NPAD = ((N + NW * CH - 1) // (NW * CH)) * (NW * CH)
PER_W = NPAD // NW
NCH = PER_W // CH
NA = N // 40
UA_PER_W = 416
UA_PAD = NW * UA_PER_W

_ridx_np = np.zeros(NPAD, np.int32)
_ridx_np[:N] = (_refs[:, 1] * W + _refs[:, 0]).astype(np.int32)
_aidx_np = np.zeros(NPAD, np.int32)
_aidx_np[:N] = (_anchors[:, 1] * W + _anchors[:, 0]).astype(np.int32)
_dx_np = np.full(NPAD, 1e6, np.float32)
_dx_np[:N] = (_anchors[:, 0] - _refs[:, 0]).astype(np.float32)
_dy_np = np.full(NPAD, 1e6, np.float32)
_dy_np[:N] = (_anchors[:, 1] - _refs[:, 1]).astype(np.float32)

_ua = _anchors[::40]
_uidx_np = np.zeros(UA_PAD, np.int32)
_uidx_np[:NA] = (_ua[:, 1] * W + _ua[:, 0]).astype(np.int32)
_uidx_np[NA:] = _uidx_np[NA - 1]

_coords8_np = np.zeros((UA_PAD, 8), np.float32)
_coords8_np[:NA, 0::2] = _ua[:, 0:1].astype(np.float32)
_coords8_np[:NA, 1::2] = _ua[:, 1:2].astype(np.float32)
_mask8_np = np.zeros((UA_PAD, 8), np.float32)
_mask8_np[:NA, 0::2] = REG_WEIGHT * 40.0
_TCROWS = UA_PAD * 8 // 128
_coords_r_np = _coords8_np.reshape(_TCROWS, 128)
_mask_r_np = _mask8_np.reshape(_TCROWS, 128)

_CONST = np.float32(4.0 * N)


_mesh = plsc.VectorSubcoreMesh(core_axis_name="c", subcore_axis_name="s")


@functools.partial(
    pl.kernel,
    out_type=(
        jax.ShapeDtypeStruct((NW, 16), jnp.float32),
        jax.ShapeDtypeStruct((UA_PAD, 8), jnp.float32),
    ),
    mesh=_mesh,
    compiler_params=pltpu.CompilerParams(
        needs_layout_passes=False, use_tc_tiling_on_sc=False),
    scratch_types=[
        pltpu.VMEM((CH,), jnp.int32),
        pltpu.VMEM((CH,), jnp.int32),
        pltpu.VMEM((CH,), jnp.float32),
        pltpu.VMEM((CH,), jnp.float32),
        pltpu.VMEM((CH, 8), jnp.float32),
        pltpu.VMEM((CH, 8), jnp.float32),
        pltpu.VMEM((16,), jnp.float32),
        pltpu.VMEM((UA_PER_W,), jnp.int32),
        pltpu.VMEM((UA_PER_W, 8), jnp.float32),
        pltpu.SemaphoreType.DMA,
        pltpu.SemaphoreType.DMA,
    ],
)
def _sc_pair_loss(table, ridx, aidx, dxs, dys, uidx, part_out, areg_out,
                  ridx_v, aidx_v, dx_v, dy_v, rrows, arows, accv,
                  uidx_v, urows, sem, sem2):
    wid = lax.axis_index("s") * 2 + lax.axis_index("c")
    base = wid * PER_W
    lane = lax.iota(jnp.int32, 16)

    acc = jnp.zeros((16,), jnp.float32)
    for k in range(NCH):
        off = base + k * CH
        pltpu.sync_copy(ridx.at[pl.ds(off, CH)], ridx_v)
        pltpu.sync_copy(aidx.at[pl.ds(off, CH)], aidx_v)
        pltpu.sync_copy(dxs.at[pl.ds(off, CH)], dx_v)
        pltpu.sync_copy(dys.at[pl.ds(off, CH)], dy_v)
        cp_r = pltpu.async_copy(table.at[ridx_v], rrows, sem)
        cp_a = pltpu.async_copy(table.at[aidx_v], arows, sem2)
        cp_r.wait()
        cp_a.wait()

        def body(j, acc):
            r0 = j * 16
            rowi = r0 + lane
            dxv = dx_v[pl.ds(r0, 16)]
            dyv = dy_v[pl.ds(r0, 16)]
            for b in range(4):
                c0 = jnp.full((16,), 2 * b, jnp.int32)
                c1 = jnp.full((16,), 2 * b + 1, jnp.int32)
                r_0 = plsc.load_gather(rrows, [rowi, c0])
                r_1 = plsc.load_gather(rrows, [rowi, c1])
                a_0 = plsc.load_gather(arows, [rowi, c0])
                a_1 = plsc.load_gather(arows, [rowi, c1])
                d0 = a_0 - r_0 + dxv
                d1 = a_1 - r_1 + dyv
                acc = acc + jnp.exp(-(d0 * d0 + d1 * d1))
            return acc

        acc = lax.fori_loop(0, CH // 16, body, acc)

    accv[...] = acc
    pltpu.sync_copy(accv, part_out.at[wid])

    ub = wid * UA_PER_W
    pltpu.sync_copy(uidx.at[pl.ds(ub, UA_PER_W)], uidx_v)
    pltpu.async_copy(table.at[uidx_v], urows, sem).wait()
    pltpu.sync_copy(urows, areg_out.at[pl.ds(ub, UA_PER_W)])


def _tc_final_body(part_ref, areg_ref, coord_ref, mask_ref, out_ref):
    a = areg_ref[...] + coord_ref[...]
    y = a * a
    ys = jnp.concatenate([y[:, 1:], y[:, :1]], axis=1)
    reg = jnp.sum(jnp.sqrt(y + ys) * mask_ref[...])
    out_ref[0, 0] = (_CONST - jnp.sum(part_ref[...])) + reg


_tc_final = pl.pallas_call(
    _tc_final_body,
    out_shape=jax.ShapeDtypeStruct((1, 1), jnp.float32),
    in_specs=[
        pl.BlockSpec(memory_space=pltpu.VMEM),
        pl.BlockSpec(memory_space=pltpu.VMEM),
        pl.BlockSpec(memory_space=pltpu.VMEM),
        pl.BlockSpec(memory_space=pltpu.VMEM),
    ],
    out_specs=pl.BlockSpec(memory_space=pltpu.SMEM),
)


def kernel(prediction):
    table = jnp.transpose(prediction, (2, 3, 0, 1)).reshape(HW, B * C)
    part, areg = _sc_pair_loss(
        table,
        jnp.asarray(_ridx_np),
        jnp.asarray(_aidx_np),
        jnp.asarray(_dx_np),
        jnp.asarray(_dy_np),
        jnp.asarray(_uidx_np),
    )
    out = _tc_final(
        part,
        areg.reshape(_TCROWS, 128),
        jnp.asarray(_coords_r_np),
        jnp.asarray(_mask_r_np),
    )
    return out[0, 0]


# speedup vs baseline: 334.8472x; 2.3810x over previous
import functools

import numpy as np
import jax
import jax.numpy as jnp
from jax import lax
from jax.experimental import pallas as pl
from jax.experimental.pallas import tpu as pltpu
from jax.experimental.pallas import tpu_sc as plsc

H = W = 512
B, C = 4, 2
HW = H * W
NPLANES = B * C
DENSITY = 0.05
KAPPA = 16
REG_WEIGHT = 0.001

NW = 32
CH = 2048
SEG = HW // NW


def _static_coords():
    rng = np.random.RandomState(0)
    num_anchors = int(DENSITY * H * W)
    ay = rng.randint(KAPPA, H - KAPPA, num_anchors)
    ax = rng.randint(KAPPA, W - KAPPA, num_anchors)
    anchors = np.stack((ax, ay), axis=1)
    num_refs = int(DENSITY * np.pi * KAPPA ** 2)
    anchors = np.repeat(anchors, num_refs, axis=0)
    n = len(anchors)
    theta = 2.0 * np.pi * rng.random_sample(n)
    r = KAPPA * rng.random_sample(n)
    offsets = np.stack((r * np.cos(theta), r * np.sin(theta)), axis=1)
    refs = (anchors + offsets).astype(np.int64)
    return anchors.astype(np.int64), refs


_anchors, _refs = _static_coords()
N = len(_anchors)
NPAD = ((N + NW * CH - 1) // (NW * CH)) * (NW * CH)
PER_W = NPAD // NW
NCH = PER_W // CH
NA = N // 40
UA_PER_W = 416
UA_PAD = NW * UA_PER_W
ATAB = 424

_ridx_np = np.zeros(NPAD, np.int32)
_ridx_np[:N] = (_refs[:, 1] * W + _refs[:, 0]).astype(np.int32)

_ua = _anchors[::40]
_uidx_np = np.zeros(UA_PAD, np.int32)
_uidx_np[:NA] = (_ua[:, 1] * W + _ua[:, 0]).astype(np.int32)
_uidx_np[NA:] = _uidx_np[NA - 1]

_MAGIC40, _SHIFT40 = 52429, 21

_wid_np = np.arange(NPAD) // PER_W
_astart_np = (((_wid_np * PER_W) // 40) // 8) * 8
_aidl_chk = np.arange(NPAD) // 40 - _astart_np
assert _aidl_chk.min() >= 0 and _aidl_chk.max() < ATAB
assert (_astart_np.max() + ATAB) <= UA_PAD
_ridx_np[N:] = _uidx_np[np.arange(N, NPAD) // 40]

_CONST = np.float32(4.0 * N - 4.0 * (NPAD - N))

_mask8_np = np.zeros((UA_PAD, 8), np.float32)
_mask8_np[:NA, 0::2] = REG_WEIGHT * 40.0
_TCROWS = UA_PAD * 8 // 128
_mask_r_np = _mask8_np.reshape(_TCROWS, 128)

_mesh = plsc.VectorSubcoreMesh(core_axis_name="c", subcore_axis_name="s")
_sc_params = pltpu.CompilerParams(
    needs_layout_passes=False, use_tc_tiling_on_sc=False)


@functools.partial(
    pl.kernel,
    out_type=jax.ShapeDtypeStruct((HW, NPLANES), jnp.float32),
    mesh=_mesh,
    compiler_params=_sc_params,
    scratch_types=[
        pltpu.VMEM((SEG,), jnp.float32),
        pltpu.VMEM((SEG,), jnp.float32),
        pltpu.VMEM((SEG, NPLANES), jnp.float32),
        pltpu.SemaphoreType.DMA,
        pltpu.SemaphoreType.DMA,
    ],
)
def _sc_interleave(pred1d, table, pb0, pb1, ilv, sm0, sm1):
    wid = lax.axis_index("s") * 2 + lax.axis_index("c")
    s0 = wid * SEG
    y0 = s0 // W
    lane = lax.iota(jnp.int32, 16)
    pbs = (pb0, pb1)
    sms = (sm0, sm1)

    cps = {0: pltpu.async_copy(pred1d.at[pl.ds(s0, SEG)], pb0, sm0)}
    for p in range(NPLANES):
        if p + 1 < NPLANES:
            cps[p + 1] = pltpu.async_copy(
                pred1d.at[pl.ds((p + 1) * HW + s0, SEG)],
                pbs[(p + 1) % 2], sms[(p + 1) % 2])
        cps[p].wait()
        pb = pbs[p % 2]
        cvec = jnp.full((16,), p, jnp.int32)
        is_x = (p % 2) == 0

        def body(jj, _, pb=pb, cvec=cvec, is_x=is_x):
            for u in range(4):
                j = jj * 4 + u
                r0 = j * 16
                v = pb[pl.ds(r0, 16)]
                if is_x:
                    coord = ((j & 31) * 16 + lane).astype(jnp.float32)
                else:
                    coord = jnp.full(
                        (16,), (y0 + (j >> 5)).astype(jnp.float32),
                        jnp.float32)
                plsc.store_scatter(ilv, [r0 + lane, cvec], v + coord)
            return 0

        lax.fori_loop(0, SEG // 64, body, 0)
    pltpu.sync_copy(ilv, table.at[pl.ds(s0, SEG)])


@functools.partial(
    pl.kernel,
    out_type=(
        jax.ShapeDtypeStruct((NW, 16), jnp.float32),
        jax.ShapeDtypeStruct((UA_PAD, NPLANES), jnp.float32),
    ),
    mesh=_mesh,
    compiler_params=_sc_params,
    scratch_types=[
        pltpu.VMEM((CH,), jnp.int32),
        pltpu.VMEM((CH,), jnp.int32),
        pltpu.VMEM((CH, NPLANES), jnp.float32),
        pltpu.VMEM((CH, NPLANES), jnp.float32),
        pltpu.VMEM((ATAB, NPLANES), jnp.float32),
        pltpu.VMEM((ATAB,), jnp.int32),
        pltpu.VMEM((16,), jnp.float32),
        pltpu.VMEM((UA_PER_W,), jnp.int32),
        pltpu.VMEM((UA_PER_W, NPLANES), jnp.float32),
        pltpu.SemaphoreType.DMA,
        pltpu.SemaphoreType.DMA,
        pltpu.SemaphoreType.DMA,
        pltpu.SemaphoreType.DMA,
        pltpu.SemaphoreType.DMA,
    ],
)
def _sc_pair_loss(table, ridx, uidx, part_out, areg_out,
                  ridx0, ridx1, rrows0, rrows1, atab, aidx_v, accv,
                  uidx_v, urows, sl0, sl1, sg0, sg1, sa):
    wid = lax.axis_index("s") * 2 + lax.axis_index("c")
    base = wid * PER_W
    astart = (((wid * PER_W) // 40) // 8) * 8
    k0 = base - astart * 40
    lane = lax.iota(jnp.int32, 16)

    pltpu.sync_copy(uidx.at[pl.ds(astart, ATAB)], aidx_v)
    cp_atab = pltpu.async_copy(table.at[aidx_v], atab, sa)

    ridxs = (ridx0, ridx1)
    rrows = (rrows0, rrows1)
    sls = (sl0, sl1)
    sgs = (sg0, sg1)

    lin = {
        k: pltpu.async_copy(
            ridx.at[pl.ds(base + k * CH, CH)], ridxs[k % 2], sls[k % 2])
        for k in range(min(2, NCH))
    }
    lin[0].wait()
    gat = {0: pltpu.async_copy(table.at[ridxs[0]], rrows0, sg0)}
    cp_atab.wait()

    acc = jnp.zeros((16,), jnp.float32)
    for k in range(NCH):
        if k + 1 < NCH:
            lin[k + 1].wait()
            gat[k + 1] = pltpu.async_copy(
                table.at[ridxs[(k + 1) % 2]], rrows[(k + 1) % 2],
                sgs[(k + 1) % 2])
        gat[k].wait()
        if k + 2 < NCH:
            lin[k + 2] = pltpu.async_copy(
                ridx.at[pl.ds(base + (k + 2) * CH, CH)],
                ridxs[k % 2], sls[k % 2])
        rr = rrows[k % 2]
        koff = k0 + k * CH

        def body(jj, accs, rr=rr, koff=koff):
            out = []
            for u in range(4):
                a = accs[u]
                r0 = (jj * 4 + u) * 16
                rowi = r0 + lane
                aidl = ((koff + r0 + lane) * _MAGIC40) >> _SHIFT40
                for b in range(B):
                    c0 = jnp.full((16,), 2 * b, jnp.int32)
                    c1 = jnp.full((16,), 2 * b + 1, jnp.int32)
                    r_0 = plsc.load_gather(rr, [rowi, c0])
                    r_1 = plsc.load_gather(rr, [rowi, c1])
                    a_0 = plsc.load_gather(atab, [aidl, c0])
                    a_1 = plsc.load_gather(atab, [aidl, c1])
                    d0 = a_0 - r_0
                    d1 = a_1 - r_1
                    a = a + jnp.exp(-(d0 * d0 + d1 * d1))
                out.append(a)
            return tuple(out)

        accs = lax.fori_loop(
            0, CH // 64, body,
            (acc, jnp.zeros((16,), jnp.float32),
             jnp.zeros((16,), jnp.float32), jnp.zeros((16,), jnp.float32)))
        acc = (accs[0] + accs[1]) + (accs[2] + accs[3])

    accv[...] = acc
    pltpu.sync_copy(accv, part_out.at[wid])

    ub = wid * UA_PER_W
    pltpu.sync_copy(uidx.at[pl.ds(ub, UA_PER_W)], uidx_v)
    pltpu.async_copy(table.at[uidx_v], urows, sa).wait()
    pltpu.sync_copy(urows, areg_out.at[pl.ds(ub, UA_PER_W)])


def _tc_final_body(part_ref, areg_ref, mask_ref, out_ref):
    y = areg_ref[...] * areg_ref[...]
    ys = jnp.concatenate([y[:, 1:], y[:, :1]], axis=1)
    reg = jnp.sum(jnp.sqrt(y + ys) * mask_ref[...])
    out_ref[0, 0] = (_CONST - jnp.sum(part_ref[...])) + reg


_tc_final = pl.pallas_call(
    _tc_final_body,
    out_shape=jax.ShapeDtypeStruct((1, 1), jnp.float32),
    in_specs=[
        pl.BlockSpec(memory_space=pltpu.VMEM),
        pl.BlockSpec(memory_space=pltpu.VMEM),
        pl.BlockSpec(memory_space=pltpu.VMEM),
    ],
    out_specs=pl.BlockSpec(memory_space=pltpu.SMEM),
)


def kernel(prediction):
    pred1d = prediction.reshape(NPLANES * HW)
    table = _sc_interleave(pred1d)
    part, areg = _sc_pair_loss(table, jnp.asarray(_ridx_np),
                               jnp.asarray(_uidx_np))
    out = _tc_final(part, areg.reshape(_TCROWS, 128), jnp.asarray(_mask_r_np))
    return out[0, 0]
